# baseline (speedup 1.0000x reference)
import functools

import jax
import jax.numpy as jnp
from jax import lax
from jax.experimental import pallas as pl
from jax.experimental.pallas import tpu as pltpu
from jax.experimental.pallas import tpu_sc as plsc

_NC = 2
_NS = 16
_NW = _NC * _NS
_CHUNK = 128


@functools.cache
def _build(V, D, N):
    per_w = N // _NW
    n_chunks = per_w // _CHUNK
    mesh = plsc.VectorSubcoreMesh(core_axis_name="c", subcore_axis_name="s")

    @functools.partial(
        pl.kernel,
        out_type=jax.ShapeDtypeStruct((N, D), jnp.float32),
        mesh=mesh,
        scratch_types=[
            pltpu.VMEM((n_chunks, _CHUNK), jnp.int32),
            pltpu.VMEM((_CHUNK, D), jnp.float32),
            pltpu.SemaphoreType.DMA,
        ],
        compiler_params=pltpu.CompilerParams(use_tc_tiling_on_sc=False),
    )
    def gather_kernel(table_hbm, idx_hbm, out_hbm, idx_v, rows_v, sem):
        wid = lax.axis_index("s") * _NC + lax.axis_index("c")
        base = wid * per_w
        pltpu.sync_copy(idx_hbm.at[wid], idx_v)

        def step(j, carry):
            pltpu.async_copy(table_hbm.at[idx_v.at[j]], rows_v, sem).wait()
            pltpu.sync_copy(rows_v, out_hbm.at[pl.ds(base + j * _CHUNK, _CHUNK)])
            return carry

        lax.fori_loop(0, n_chunks, step, 0)

    return gather_kernel


def kernel(x, embedding_weight):
    B, H = x.shape
    V, D = embedding_weight.shape
    N = B * H
    idx = x.reshape(_NW, N // _NW // _CHUNK, _CHUNK).astype(jnp.int32)
    out = _build(V, D, N)(embedding_weight, idx)
    return out.reshape(B, H, D)


# speedup vs baseline: 1.1169x; 1.1169x over previous
import functools

import jax
import jax.numpy as jnp
from jax import lax
from jax.experimental import pallas as pl
from jax.experimental.pallas import tpu as pltpu
from jax.experimental.pallas import tpu_sc as plsc

_NC = 2
_NS = 16
_NW = _NC * _NS
_CHUNK = 128
_K = 4
_SUPER = _CHUNK * _K


@functools.cache
def _build(V, D, N):
    per_w = N // _NW
    n_chunks = per_w // _CHUNK
    n_super = n_chunks // _K
    mesh = plsc.VectorSubcoreMesh(core_axis_name="c", subcore_axis_name="s")

    @functools.partial(
        pl.kernel,
        out_type=jax.ShapeDtypeStruct((N, D), jnp.float32),
        mesh=mesh,
        scratch_types=[
            pltpu.VMEM((n_chunks, _CHUNK), jnp.int32),
            pltpu.VMEM((2, _SUPER, D), jnp.float32),
            pltpu.SemaphoreType.DMA((2,)),
            pltpu.SemaphoreType.DMA((2,)),
        ],
        compiler_params=pltpu.CompilerParams(use_tc_tiling_on_sc=False),
    )
    def gather_kernel(table_hbm, idx_hbm, out_hbm, idx_v, rows_v, gsem, wsem):
        wid = lax.axis_index("s") * _NC + lax.axis_index("c")
        base = wid * per_w
        pltpu.sync_copy(idx_hbm.at[wid], idx_v)

        def issue_gathers(g, b):
            for k in range(_K):
                pltpu.async_copy(
                    table_hbm.at[idx_v.at[g * _K + k]],
                    rows_v.at[b, pl.ds(k * _CHUNK, _CHUNK)],
                    gsem.at[b],
                )

        def wait_gathers(b):
            for k in range(_K):
                pltpu.make_async_copy(
                    table_hbm.at[idx_v.at[0]],
                    rows_v.at[b, pl.ds(k * _CHUNK, _CHUNK)],
                    gsem.at[b],
                ).wait()

        def wait_write(b):
            pltpu.make_async_copy(
                rows_v.at[b],
                out_hbm.at[pl.ds(base, _SUPER)],
                wsem.at[b],
            ).wait()

        issue_gathers(0, 0)
        issue_gathers(1, 1)

        def step(g, carry):
            b = lax.rem(g, 2)
            wait_gathers(b)
            pltpu.async_copy(
                rows_v.at[b],
                out_hbm.at[pl.ds(base + g * _SUPER, _SUPER)],
                wsem.at[b],
            )

            @pl.when(g + 2 < n_super)
            def _():
                wait_write(b)
                issue_gathers(g + 2, b)

            return carry

        lax.fori_loop(0, n_super, step, 0)
        wait_write(0)
        wait_write(1)

    return gather_kernel


def kernel(x, embedding_weight):
    B, H = x.shape
    V, D = embedding_weight.shape
    N = B * H
    idx = x.reshape(_NW, N // _NW // _CHUNK, _CHUNK).astype(jnp.int32)
    out = _build(V, D, N)(embedding_weight, idx)
    return out.reshape(B, H, D)


# speedup vs baseline: 1.4836x; 1.3283x over previous
import functools

import jax
import jax.numpy as jnp
from jax import lax
from jax.experimental import pallas as pl
from jax.experimental.pallas import tpu as pltpu
from jax.experimental.pallas import tpu_sc as plsc

_NC = 2
_NS = 16
_NW = _NC * _NS
_CHUNK = 128
_K = 4
_SUPER = _CHUNK * _K


@functools.cache
def _build(V, D, N):
    per_w = N // _NW
    n_chunks = per_w // _CHUNK
    n_super = n_chunks // _K
    mesh = plsc.VectorSubcoreMesh(core_axis_name="c", subcore_axis_name="s")

    @functools.partial(
        pl.kernel,
        out_type=jax.ShapeDtypeStruct((N, 2 * D), jnp.float32),
        mesh=mesh,
        scratch_types=[
            pltpu.VMEM((n_chunks, _CHUNK), jnp.int32),
            pltpu.VMEM((2, _SUPER, D), jnp.float32),
            pltpu.SemaphoreType.DMA((2,)),
            pltpu.SemaphoreType.DMA((2,)),
        ],
        compiler_params=pltpu.CompilerParams(use_tc_tiling_on_sc=False),
    )
    def gather_kernel(table_hbm, idx_hbm, out_hbm, idx_v, rows_v, gsem, wsem):
        wid = lax.axis_index("s") * _NC + lax.axis_index("c")
        base = wid * per_w
        pltpu.sync_copy(idx_hbm.at[wid], idx_v)

        def issue_gathers(g, b):
            for k in range(_K):
                pltpu.async_copy(
                    table_hbm.at[idx_v.at[g * _K + k]],
                    rows_v.at[b, pl.ds(k * _CHUNK, _CHUNK)],
                    gsem.at[b],
                )

        def wait_gathers(b):
            for k in range(_K):
                pltpu.make_async_copy(
                    table_hbm.at[idx_v.at[0]],
                    rows_v.at[b, pl.ds(k * _CHUNK, _CHUNK)],
                    gsem.at[b],
                ).wait()

        def wait_write(b):
            pltpu.make_async_copy(
                rows_v.at[b],
                out_hbm.at[pl.ds(base, _SUPER), pl.ds(0, D)],
                wsem.at[b],
            ).wait()

        issue_gathers(0, 0)
        issue_gathers(1, 1)

        def step(g, carry):
            b = lax.rem(g, 2)
            wait_gathers(b)
            pltpu.async_copy(
                rows_v.at[b],
                out_hbm.at[pl.ds(base + g * _SUPER, _SUPER), pl.ds(0, D)],
                wsem.at[b],
            )

            @pl.when(g + 2 < n_super)
            def _():
                wait_write(b)
                issue_gathers(g + 2, b)

            return carry

        lax.fori_loop(0, n_super, step, 0)
        wait_write(0)
        wait_write(1)

    return gather_kernel


def kernel(x, embedding_weight):
    B, H = x.shape
    V, D = embedding_weight.shape
    N = B * H
    idx = x.reshape(_NW, N // _NW // _CHUNK, _CHUNK).astype(jnp.int32)
    out = _build(V, D, N)(embedding_weight, idx)
    return out[:, :D].reshape(B, H, D)


# speedup vs baseline: 1.4853x; 1.0012x over previous
import functools

import jax
import jax.numpy as jnp
from jax import lax
from jax.experimental import pallas as pl
from jax.experimental.pallas import tpu as pltpu
from jax.experimental.pallas import tpu_sc as plsc

_NC = 2
_NS = 16
_NW = _NC * _NS
_CHUNK = 128
_K = 4
_SUPER = _CHUNK * _K
_NBUF = 3


@functools.cache
def _build(V, D, N):
    per_w = N // _NW
    n_chunks = per_w // _CHUNK
    n_super = n_chunks // _K
    mesh = plsc.VectorSubcoreMesh(core_axis_name="c", subcore_axis_name="s")

    @functools.partial(
        pl.kernel,
        out_type=jax.ShapeDtypeStruct((N, 2 * D), jnp.float32),
        mesh=mesh,
        scratch_types=[
            pltpu.VMEM((n_chunks, _CHUNK), jnp.int32),
            pltpu.VMEM((_NBUF, _SUPER, D), jnp.float32),
            pltpu.SemaphoreType.DMA((_NBUF,)),
            pltpu.SemaphoreType.DMA((_NBUF,)),
        ],
        compiler_params=pltpu.CompilerParams(use_tc_tiling_on_sc=False),
    )
    def gather_kernel(table_hbm, idx_hbm, out_hbm, idx_v, rows_v, gsem, wsem):
        wid = lax.axis_index("s") * _NC + lax.axis_index("c")
        base = wid * per_w
        pltpu.sync_copy(idx_hbm.at[wid], idx_v)

        def issue_gathers(g, b):
            for k in range(_K):
                pltpu.async_copy(
                    table_hbm.at[idx_v.at[g * _K + k]],
                    rows_v.at[b, pl.ds(k * _CHUNK, _CHUNK)],
                    gsem.at[b],
                )

        def wait_gathers(b):
            for k in range(_K):
                pltpu.make_async_copy(
                    table_hbm.at[idx_v.at[0]],
                    rows_v.at[b, pl.ds(k * _CHUNK, _CHUNK)],
                    gsem.at[b],
                ).wait()

        def wait_write(b):
            pltpu.make_async_copy(
                rows_v.at[b],
                out_hbm.at[pl.ds(base, _SUPER), pl.ds(0, D)],
                wsem.at[b],
            ).wait()

        for b in range(_NBUF):
            issue_gathers(b, b)

        def step(g, carry):
            b = lax.rem(g, _NBUF)
            wait_gathers(b)
            pltpu.async_copy(
                rows_v.at[b],
                out_hbm.at[pl.ds(base + g * _SUPER, _SUPER), pl.ds(0, D)],
                wsem.at[b],
            )

            @pl.when(g + _NBUF < n_super)
            def _():
                wait_write(b)
                issue_gathers(g + _NBUF, b)

            return carry

        lax.fori_loop(0, n_super, step, 0)
        for b in range(_NBUF):
            wait_write(b)

    return gather_kernel


def kernel(x, embedding_weight):
    B, H = x.shape
    V, D = embedding_weight.shape
    N = B * H
    idx = x.reshape(_NW, N // _NW // _CHUNK, _CHUNK).astype(jnp.int32)
    out = _build(V, D, N)(embedding_weight, idx)
    return out[:, :D].reshape(B, H, D)
